# Initial kernel scaffold; baseline (speedup 1.0000x reference)
import jax
import jax.numpy as jnp
from jax.experimental import pallas as pl


def kernel(weight, scores):
    raise NotImplementedError("write your pallas kernel here")


# speedup vs baseline: 21.1292x; 21.1292x over previous
import functools

import jax
import jax.numpy as jnp
from jax import lax
from jax.experimental import pallas as pl
from jax.experimental.pallas import tpu as pltpu
from jax.experimental.pallas import tpu_sc as plsc

NC = 2
NS = 16
NW = NC * NS
CHUNK = 16384

_MIN_I32 = -(2 ** 31)


def _shiftr(x, amount):
    if amount == 0:
        return x
    return lax.shift_right_logical(x, jnp.full((16,), amount, jnp.int32))


def _hist_body(nb, shift, check_shift, has_state, n_per_w, *refs):
    if has_state:
        scores_hbm, state_hbm, out_hbm, buf, lhist, merged, staterow = refs
    else:
        scores_hbm, out_hbm, buf, lhist, merged = refs
    c = lax.axis_index("c")
    s = lax.axis_index("s")
    wid = s * NC + c
    base = wid * n_per_w
    lane = lax.iota(jnp.int32, 16)
    ones = lane * 0 + 1
    zeros = lane * 0

    def zero_body(i, _):
        lhist[pl.ds(i * 16, 16)] = zeros
        return 0

    lax.fori_loop(0, nb, zero_body, 0)

    if has_state:
        pltpu.sync_copy(state_hbm.at[0], staterow)
        prefix = staterow[pl.ds(0, 16)]

    def chunk_body(ch, _):
        pltpu.sync_copy(scores_hbm.at[pl.ds(base + ch * CHUNK, CHUNK)], buf)

        def inner(i, _):
            b = buf[pl.ds(i * 16, 16)]
            key = b ^ ((b >> 31) | _MIN_I32)
            bucket = _shiftr(key, shift) & (nb - 1)
            idx = lane * nb + bucket
            if has_state:
                valid = _shiftr(key, check_shift) == prefix
                plsc.addupdate_scatter(lhist, [idx], ones, mask=valid)
            else:
                plsc.addupdate_scatter(lhist, [idx], ones)
            return 0

        lax.fori_loop(0, CHUNK // 16, inner, 0)
        return 0

    lax.fori_loop(0, n_per_w // CHUNK, chunk_body, 0)

    def merge_body(j, _):
        acc = lhist[pl.ds(j * 16, 16)]
        for l in range(1, 16):
            acc = acc + lhist[pl.ds(l * nb + j * 16, 16)]
        merged[pl.ds(j * 16, 16)] = acc
        return 0

    lax.fori_loop(0, nb // 16, merge_body, 0)
    pltpu.sync_copy(merged, out_hbm.at[wid])


def _hist_pass(flat, nb, shift, check_shift, state):
    n_per_w = flat.shape[0] // NW
    mesh = plsc.VectorSubcoreMesh(core_axis_name="c", subcore_axis_name="s",
                                  num_cores=NC, num_subcores=NS)
    scratch = [
        pltpu.VMEM((CHUNK,), jnp.int32),
        pltpu.VMEM((16 * nb,), jnp.int32),
        pltpu.VMEM((nb,), jnp.int32),
    ]
    if state is not None:
        scratch.append(pltpu.VMEM((128,), jnp.int32))
    body = functools.partial(_hist_body, nb, shift, check_shift,
                             state is not None, n_per_w)
    kfn = pl.kernel(
        body,
        out_type=jax.ShapeDtypeStruct((NW, nb), jnp.int32),
        mesh=mesh,
        scratch_types=scratch,
        compiler_params=pltpu.CompilerParams(needs_layout_passes=False),
    )
    return kfn(flat, state) if state is not None else kfn(flat)


def _select_body(nb, mult, k_static, has_state, hist_ref, *rest):
    if has_state:
        state_ref, out_ref = rest
    else:
        (out_ref,) = rest
    r = nb // 128
    h3 = hist_ref[...].astype(jnp.float32)
    hist = jnp.sum(h3, axis=0)
    li = lax.broadcasted_iota(jnp.int32, (128, 128), 0)
    lj = lax.broadcasted_iota(jnp.int32, (128, 128), 1)
    ltri = (li <= lj).astype(jnp.float32)
    cum_in = jnp.dot(hist, ltri, preferred_element_type=jnp.float32,
                     precision=lax.Precision.HIGHEST)
    row_tot = cum_in[:, 127:128]
    si = lax.broadcasted_iota(jnp.int32, (r, r), 0)
    sj = lax.broadcasted_iota(jnp.int32, (r, r), 1)
    stri = (sj < si).astype(jnp.float32)
    off = jnp.dot(stri, row_tot, preferred_element_type=jnp.float32,
                  precision=lax.Precision.HIGHEST)
    cum = cum_in + off
    if has_state:
        prefix = state_ref[0, 0]
        kr = state_ref[1, 0].astype(jnp.float32)
    else:
        prefix = jnp.int32(0)
        kr = jnp.float32(k_static)
    jstar = jnp.sum((cum < kr).astype(jnp.int32))
    excl = cum - hist
    fi = (128 * lax.broadcasted_iota(jnp.int32, (r, 128), 0)
          + lax.broadcasted_iota(jnp.int32, (r, 128), 1))
    excl_at = jnp.sum(jnp.where(fi == jstar, excl, 0.0))
    new_kr = (kr - excl_at).astype(jnp.int32)
    new_prefix = prefix * mult + jstar
    ri = lax.broadcasted_iota(jnp.int32, (8, 128), 0)
    out_ref[...] = jnp.where(ri == 0, new_prefix,
                             jnp.where(ri == 1, new_kr, 0))


def _select(hist, nb, mult, k_static, state):
    r = nb // 128
    h3 = hist.reshape(NW, r, 128)
    args = (h3,) if state is None else (h3, state)
    return pl.pallas_call(
        functools.partial(_select_body, nb, mult, k_static, state is not None),
        out_shape=jax.ShapeDtypeStruct((8, 128), jnp.int32),
    )(*args)


def _mask_body(key_ref, w_ref, s_ref, o_ref):
    key = key_ref[0, 0]
    bits = jnp.where(key < 0, key & jnp.int32(0x7FFFFFFF), ~key)
    bits_v = jnp.zeros((1, 1), jnp.int32) + bits
    thr = lax.bitcast_convert_type(bits_v, jnp.float32)
    o_ref[...] = jnp.where(s_ref[...] < thr, 0.0, w_ref[...])


def _mask(key, weight, scores):
    rows, cols = scores.shape
    blk = 128
    return pl.pallas_call(
        _mask_body,
        grid=(rows // blk,),
        in_specs=[
            pl.BlockSpec(memory_space=pltpu.SMEM),
            pl.BlockSpec((blk, cols), lambda i: (i, 0)),
            pl.BlockSpec((blk, cols), lambda i: (i, 0)),
        ],
        out_specs=pl.BlockSpec((blk, cols), lambda i: (i, 0)),
        out_shape=jax.ShapeDtypeStruct(scores.shape, jnp.float32),
    )(key, weight, scores)


def kernel(weight, scores):
    n = scores.size
    k = int(1 + round(0.9 * (n - 1)))
    flat = lax.bitcast_convert_type(scores.reshape(-1), jnp.int32)
    h1 = _hist_pass(flat, 4096, 20, None, None)
    st1 = _select(h1, 4096, 1, k, None)
    h2 = _hist_pass(flat, 4096, 8, 20, st1)
    st2 = _select(h2, 4096, 4096, None, st1)
    h3 = _hist_pass(flat, 256, 0, 8, st2)
    st3 = _select(h3, 256, 256, None, st2)
    key = lax.slice(st3, (0, 0), (1, 1))
    return _mask(key, weight, scores)


# speedup vs baseline: 93.5934x; 4.4296x over previous
import functools

import jax
import jax.numpy as jnp
from jax import lax
from jax.experimental import pallas as pl
from jax.experimental.pallas import tpu as pltpu
from jax.experimental.pallas import tpu_sc as plsc

NC = 2
NS = 16
NW = NC * NS
NB = 65536
CHUNK = 16384

_MIN_I32 = -(2 ** 31)


def _shiftr16(x):
    return lax.shift_right_logical(x, jnp.full((16,), 16, jnp.int32))


def _hist_body(first_pass, n_per_w, *refs):
    if first_pass:
        scores_hbm, out_hbm, buf_a, buf_b, hist, sem_a, sem_b = refs
    else:
        (scores_hbm, state_hbm, out_hbm, buf_a, buf_b, hist, staterow,
         sem_a, sem_b) = refs
    c = lax.axis_index("c")
    s = lax.axis_index("s")
    wid = s * NC + c
    base = wid * n_per_w
    lane = lax.iota(jnp.int32, 16)
    zeros = lane * 0
    ones = zeros + 1

    @plsc.parallel_loop(0, NB // 16, 1, unroll=8)
    def _(i):
        hist[pl.ds(i * 16, 16)] = zeros

    if not first_pass:
        pltpu.sync_copy(state_hbm.at[0], staterow)
        prefix = staterow[pl.ds(0, 16)]

    bufs = (buf_a, buf_b)
    sems = (sem_a, sem_b)
    n_chunks = n_per_w // CHUNK

    def chunk_src(ch):
        return scores_hbm.at[pl.ds(base + ch * CHUNK, CHUNK)]

    def compute(buf):
        @plsc.parallel_loop(0, CHUNK // 16, 1, unroll=8)
        def _(i):
            b = buf[pl.ds(i * 16, 16)]
            key = b ^ ((b >> 31) | _MIN_I32)
            if first_pass:
                plsc.addupdate_scatter(hist, [_shiftr16(key)], ones)
            else:
                valid = _shiftr16(key) == prefix
                plsc.addupdate_scatter(hist, [key & 0xFFFF], ones,
                                       mask=valid)

    descs = [None, None]
    descs[0] = pltpu.async_copy(chunk_src(0), bufs[0], sems[0])
    for ch in range(n_chunks):
        if ch + 1 < n_chunks:
            descs[(ch + 1) % 2] = pltpu.async_copy(
                chunk_src(ch + 1), bufs[(ch + 1) % 2], sems[(ch + 1) % 2])
        descs[ch % 2].wait()
        compute(bufs[ch % 2])

    pltpu.sync_copy(hist, out_hbm.at[wid])


def _hist_pass(flat, state):
    n_per_w = flat.shape[0] // NW
    mesh = plsc.VectorSubcoreMesh(core_axis_name="c", subcore_axis_name="s",
                                  num_cores=NC, num_subcores=NS)
    scratch = [
        pltpu.VMEM((CHUNK,), jnp.int32),
        pltpu.VMEM((CHUNK,), jnp.int32),
        pltpu.VMEM((NB,), jnp.int32),
    ]
    if state is not None:
        scratch.append(pltpu.VMEM((128,), jnp.int32))
    scratch.extend([pltpu.SemaphoreType.DMA, pltpu.SemaphoreType.DMA])
    body = functools.partial(_hist_body, state is None, n_per_w)
    kfn = pl.kernel(
        body,
        out_type=jax.ShapeDtypeStruct((NW, NB), jnp.int32),
        mesh=mesh,
        scratch_types=scratch,
        compiler_params=pltpu.CompilerParams(needs_layout_passes=False),
    )
    return kfn(flat) if state is None else kfn(flat, state)


def _select_body(mult, k_static, has_state, hist_ref, *rest):
    if has_state:
        state_ref, out_ref = rest
    else:
        (out_ref,) = rest
    r = NB // 128
    h3 = hist_ref[...].astype(jnp.float32)
    hist = jnp.sum(h3, axis=0)
    li = lax.broadcasted_iota(jnp.int32, (128, 128), 0)
    lj = lax.broadcasted_iota(jnp.int32, (128, 128), 1)
    ltri = (li <= lj).astype(jnp.float32)
    cum_in = jnp.dot(hist, ltri, preferred_element_type=jnp.float32,
                     precision=lax.Precision.HIGHEST)
    row_tot = cum_in[:, 127:128]
    si = lax.broadcasted_iota(jnp.int32, (r, r), 0)
    sj = lax.broadcasted_iota(jnp.int32, (r, r), 1)
    stri = (sj < si).astype(jnp.float32)
    off = jnp.dot(stri, row_tot, preferred_element_type=jnp.float32,
                  precision=lax.Precision.HIGHEST)
    cum = cum_in + off
    if has_state:
        prefix = state_ref[0, 0]
        kr = state_ref[1, 0].astype(jnp.float32)
    else:
        prefix = jnp.int32(0)
        kr = jnp.float32(k_static)
    jstar = jnp.sum((cum < kr).astype(jnp.int32))
    excl = cum - hist
    fi = (128 * lax.broadcasted_iota(jnp.int32, (r, 128), 0)
          + lax.broadcasted_iota(jnp.int32, (r, 128), 1))
    excl_at = jnp.sum(jnp.where(fi == jstar, excl, 0.0))
    new_kr = (kr - excl_at).astype(jnp.int32)
    new_prefix = prefix * mult + jstar
    ri = lax.broadcasted_iota(jnp.int32, (8, 128), 0)
    out_ref[...] = jnp.where(ri == 0, new_prefix,
                             jnp.where(ri == 1, new_kr, 0))


def _select(hist, mult, k_static, state):
    r = NB // 128
    h3 = hist.reshape(NW, r, 128)
    args = (h3,) if state is None else (h3, state)
    return pl.pallas_call(
        functools.partial(_select_body, mult, k_static, state is not None),
        out_shape=jax.ShapeDtypeStruct((8, 128), jnp.int32),
    )(*args)


def _mask_body(key_ref, w_ref, s_ref, o_ref):
    key = key_ref[0, 0]
    bits = jnp.where(key < 0, key & jnp.int32(0x7FFFFFFF), ~key)
    bits_v = jnp.zeros((1, 1), jnp.int32) + bits
    thr = lax.bitcast_convert_type(bits_v, jnp.float32)
    o_ref[...] = jnp.where(s_ref[...] < thr, 0.0, w_ref[...])


def _mask(key, weight, scores):
    rows, cols = scores.shape
    blk = 128
    return pl.pallas_call(
        _mask_body,
        grid=(rows // blk,),
        in_specs=[
            pl.BlockSpec(memory_space=pltpu.SMEM),
            pl.BlockSpec((blk, cols), lambda i: (i, 0)),
            pl.BlockSpec((blk, cols), lambda i: (i, 0)),
        ],
        out_specs=pl.BlockSpec((blk, cols), lambda i: (i, 0)),
        out_shape=jax.ShapeDtypeStruct(scores.shape, jnp.float32),
    )(key, weight, scores)


def kernel(weight, scores):
    n = scores.size
    k = int(1 + round(0.9 * (n - 1)))
    flat = lax.bitcast_convert_type(scores.reshape(-1), jnp.int32)
    h1 = _hist_pass(flat, None)
    st1 = _select(h1, 1, k, None)
    h2 = _hist_pass(flat, st1)
    st2 = _select(h2, NB, None, st1)
    key = lax.slice(st2, (0, 0), (1, 1))
    return _mask(key, weight, scores)


# speedup vs baseline: 106.4349x; 1.1372x over previous
import functools

import jax
import jax.numpy as jnp
from jax import lax
from jax.experimental import pallas as pl
from jax.experimental.pallas import tpu as pltpu
from jax.experimental.pallas import tpu_sc as plsc

NC = 2
NS = 16
NW = NC * NS
NB = 65536
CHUNK = 16384

_MIN_I32 = -(2 ** 31)


def _shiftr16(x):
    return lax.shift_right_logical(x, jnp.full((16,), 16, jnp.int32))


def _hist_body(first_pass, n_per_w, *refs):
    if first_pass:
        scores_hbm, out_hbm, buf_a, buf_b, hist, sem_a, sem_b = refs
    else:
        (scores_hbm, state_hbm, out_hbm, buf_a, buf_b, hist, staterow,
         sem_a, sem_b) = refs
    c = lax.axis_index("c")
    s = lax.axis_index("s")
    wid = s * NC + c
    base = wid * n_per_w
    lane = lax.iota(jnp.int32, 16)
    zeros = lane * 0
    ones = zeros + 1

    @plsc.parallel_loop(0, NB // 16, 1, unroll=8)
    def _(i):
        hist[pl.ds(i * 16, 16)] = zeros

    if not first_pass:
        pltpu.sync_copy(state_hbm.at[0], staterow)
        prefix = staterow[pl.ds(0, 16)]

    bufs = (buf_a, buf_b)
    sems = (sem_a, sem_b)
    n_chunks = n_per_w // CHUNK

    def chunk_src(ch):
        return scores_hbm.at[pl.ds(base + ch * CHUNK, CHUNK)]

    def compute(buf):
        @plsc.parallel_loop(0, CHUNK // 16, 1, unroll=8)
        def _(i):
            b = plsc.bitcast(buf[pl.ds(i * 16, 16)], jnp.int32)
            key = b ^ ((b >> 31) | _MIN_I32)
            if first_pass:
                plsc.addupdate_scatter(hist, [_shiftr16(key)], ones)
            else:
                valid = _shiftr16(key) == prefix
                plsc.addupdate_scatter(hist, [key & 0xFFFF], ones,
                                       mask=valid)

    descs = [None, None]
    descs[0] = pltpu.async_copy(chunk_src(0), bufs[0], sems[0])
    for ch in range(n_chunks):
        if ch + 1 < n_chunks:
            descs[(ch + 1) % 2] = pltpu.async_copy(
                chunk_src(ch + 1), bufs[(ch + 1) % 2], sems[(ch + 1) % 2])
        descs[ch % 2].wait()
        compute(bufs[ch % 2])

    pltpu.sync_copy(hist, out_hbm.at[wid])


def _hist_pass(flat, state):
    n_per_w = flat.shape[0] // NW
    mesh = plsc.VectorSubcoreMesh(core_axis_name="c", subcore_axis_name="s",
                                  num_cores=NC, num_subcores=NS)
    scratch = [
        pltpu.VMEM((CHUNK,), jnp.float32),
        pltpu.VMEM((CHUNK,), jnp.float32),
        pltpu.VMEM((NB,), jnp.int32),
    ]
    if state is not None:
        scratch.append(pltpu.VMEM((128,), jnp.int32))
    scratch.extend([pltpu.SemaphoreType.DMA, pltpu.SemaphoreType.DMA])
    body = functools.partial(_hist_body, state is None, n_per_w)
    kfn = pl.kernel(
        body,
        out_type=jax.ShapeDtypeStruct((NW, NB), jnp.int32),
        mesh=mesh,
        scratch_types=scratch,
        compiler_params=pltpu.CompilerParams(needs_layout_passes=False),
    )
    return kfn(flat) if state is None else kfn(flat, state)


def _select_body(mult, k_static, has_state, hist_ref, *rest):
    if has_state:
        state_ref, out_ref = rest
    else:
        (out_ref,) = rest
    r = NB // 128
    h3 = hist_ref[...].astype(jnp.float32)
    hist = jnp.sum(h3, axis=0)
    li = lax.broadcasted_iota(jnp.int32, (128, 128), 0)
    lj = lax.broadcasted_iota(jnp.int32, (128, 128), 1)
    ltri = (li <= lj).astype(jnp.float32)
    cum_in = jnp.dot(hist, ltri, preferred_element_type=jnp.float32,
                     precision=lax.Precision.HIGHEST)
    row_tot = cum_in[:, 127:128]
    si = lax.broadcasted_iota(jnp.int32, (r, r), 0)
    sj = lax.broadcasted_iota(jnp.int32, (r, r), 1)
    stri = (sj < si).astype(jnp.float32)
    off = jnp.dot(stri, row_tot, preferred_element_type=jnp.float32,
                  precision=lax.Precision.HIGHEST)
    cum = cum_in + off
    if has_state:
        prefix = state_ref[0, 0]
        kr = state_ref[1, 0].astype(jnp.float32)
    else:
        prefix = jnp.int32(0)
        kr = jnp.float32(k_static)
    jstar = jnp.sum((cum < kr).astype(jnp.int32))
    excl = cum - hist
    fi = (128 * lax.broadcasted_iota(jnp.int32, (r, 128), 0)
          + lax.broadcasted_iota(jnp.int32, (r, 128), 1))
    excl_at = jnp.sum(jnp.where(fi == jstar, excl, 0.0))
    new_kr = (kr - excl_at).astype(jnp.int32)
    new_prefix = prefix * mult + jstar
    ri = lax.broadcasted_iota(jnp.int32, (8, 128), 0)
    out_ref[...] = jnp.where(ri == 0, new_prefix,
                             jnp.where(ri == 1, new_kr, 0))


def _select(hist, mult, k_static, state):
    r = NB // 128
    h3 = hist.reshape(NW, r, 128)
    args = (h3,) if state is None else (h3, state)
    return pl.pallas_call(
        functools.partial(_select_body, mult, k_static, state is not None),
        out_shape=jax.ShapeDtypeStruct((8, 128), jnp.int32),
    )(*args)


def _mask_body(key_ref, w_ref, s_ref, o_ref):
    key = key_ref[0, 0]
    bits = jnp.where(key < 0, key & jnp.int32(0x7FFFFFFF), ~key)
    bits_v = jnp.zeros((1, 1), jnp.int32) + bits
    thr = lax.bitcast_convert_type(bits_v, jnp.float32)
    o_ref[...] = jnp.where(s_ref[...] < thr, 0.0, w_ref[...])


def _mask(key, weight, scores):
    rows, cols = scores.shape
    blk = 128
    return pl.pallas_call(
        _mask_body,
        grid=(rows // blk,),
        in_specs=[
            pl.BlockSpec(memory_space=pltpu.SMEM),
            pl.BlockSpec((blk, cols), lambda i: (i, 0)),
            pl.BlockSpec((blk, cols), lambda i: (i, 0)),
        ],
        out_specs=pl.BlockSpec((blk, cols), lambda i: (i, 0)),
        out_shape=jax.ShapeDtypeStruct(scores.shape, jnp.float32),
    )(key, weight, scores)


def kernel(weight, scores):
    n = scores.size
    k = int(1 + round(0.9 * (n - 1)))
    flat = scores.reshape(-1)
    h1 = _hist_pass(flat, None)
    st1 = _select(h1, 1, k, None)
    h2 = _hist_pass(flat, st1)
    st2 = _select(h2, NB, None, st1)
    key = lax.slice(st2, (0, 0), (1, 1))
    return _mask(key, weight, scores)


# speedup vs baseline: 143.1356x; 1.3448x over previous
import functools

import jax
import jax.numpy as jnp
from jax import lax
from jax.experimental import pallas as pl
from jax.experimental.pallas import tpu as pltpu
from jax.experimental.pallas import tpu_sc as plsc

NC = 2
NS = 16
NW = NC * NS
NB = 65536
CHUNK = 16384

_MIN_I32 = -(2 ** 31)


def _shiftr16(x):
    return lax.shift_right_logical(x, jnp.full((16,), 16, jnp.int32))


def _hist_body(first_pass, rows_per_w, *refs):
    if first_pass:
        scores_hbm, out_hbm, buf_a, buf_b, hist, sem_a, sem_b = refs
    else:
        (scores_hbm, state_hbm, out_hbm, buf_a, buf_b, hist, staterow,
         sem_a, sem_b) = refs
    c = lax.axis_index("c")
    s = lax.axis_index("s")
    wid = s * NC + c
    base_row = wid * rows_per_w
    lane = lax.iota(jnp.int32, 16)
    zeros = lane * 0
    ones = zeros + 1

    @plsc.parallel_loop(0, NB // 16, 1, unroll=8)
    def _(i):
        hist[pl.ds(i * 16, 16)] = zeros

    if not first_pass:
        pltpu.sync_copy(state_hbm.at[0], staterow)
        prefix = staterow[pl.ds(0, 16)]

    bufs = (buf_a, buf_b)
    sems = (sem_a, sem_b)
    n_chunks = 2 * (rows_per_w // 8)

    def chunk_src(ch):
        return scores_hbm.at[pl.ds(base_row + (ch // 2) * 8, 8),
                             pl.ds((ch % 2) * 2048, 2048)]

    def compute(buf):
        @plsc.parallel_loop(0, CHUNK // 16, 1, unroll=8)
        def _(i):
            b = plsc.bitcast(buf[i >> 7, pl.ds((i & 127) * 16, 16)],
                             jnp.int32)
            key = b ^ ((b >> 31) | _MIN_I32)
            if first_pass:
                plsc.addupdate_scatter(hist, [_shiftr16(key)], ones)
            else:
                valid = _shiftr16(key) == prefix
                plsc.addupdate_scatter(hist, [key & 0xFFFF], ones,
                                       mask=valid)

    descs = [None, None]
    descs[0] = pltpu.async_copy(chunk_src(0), bufs[0], sems[0])
    for ch in range(n_chunks):
        if ch + 1 < n_chunks:
            descs[(ch + 1) % 2] = pltpu.async_copy(
                chunk_src(ch + 1), bufs[(ch + 1) % 2], sems[(ch + 1) % 2])
        descs[ch % 2].wait()
        compute(bufs[ch % 2])

    pltpu.sync_copy(hist, out_hbm.at[pl.ds(wid * NB, NB)])


def _hist_pass(scores, state):
    rows_per_w = scores.shape[0] // NW
    mesh = plsc.VectorSubcoreMesh(core_axis_name="c", subcore_axis_name="s",
                                  num_cores=NC, num_subcores=NS)
    scratch = [
        pltpu.VMEM((8, 2048), jnp.float32),
        pltpu.VMEM((8, 2048), jnp.float32),
        pltpu.VMEM((NB,), jnp.int32),
    ]
    if state is not None:
        scratch.append(pltpu.VMEM((128,), jnp.int32))
    scratch.extend([pltpu.SemaphoreType.DMA, pltpu.SemaphoreType.DMA])
    body = functools.partial(_hist_body, state is None, rows_per_w)
    kfn = pl.kernel(
        body,
        out_type=jax.ShapeDtypeStruct((NW * NB,), jnp.int32),
        mesh=mesh,
        scratch_types=scratch,
        compiler_params=pltpu.CompilerParams(needs_layout_passes=False,
                                             use_tc_tiling_on_sc=True),
    )
    return kfn(scores) if state is None else kfn(scores, state)


def _select_body(mult, k_static, has_state, hist_ref, *rest):
    if has_state:
        state_ref, out_ref = rest
    else:
        (out_ref,) = rest
    r = NB // 128
    h3 = hist_ref[...].astype(jnp.float32)
    hist = jnp.sum(h3, axis=0)
    li = lax.broadcasted_iota(jnp.int32, (128, 128), 0)
    lj = lax.broadcasted_iota(jnp.int32, (128, 128), 1)
    ltri = (li <= lj).astype(jnp.float32)
    cum_in = jnp.dot(hist, ltri, preferred_element_type=jnp.float32,
                     precision=lax.Precision.HIGHEST)
    row_tot = cum_in[:, 127:128]
    si = lax.broadcasted_iota(jnp.int32, (r, r), 0)
    sj = lax.broadcasted_iota(jnp.int32, (r, r), 1)
    stri = (sj < si).astype(jnp.float32)
    off = jnp.dot(stri, row_tot, preferred_element_type=jnp.float32,
                  precision=lax.Precision.HIGHEST)
    cum = cum_in + off
    if has_state:
        prefix = state_ref[0, 0]
        kr = state_ref[1, 0].astype(jnp.float32)
    else:
        prefix = jnp.int32(0)
        kr = jnp.float32(k_static)
    jstar = jnp.sum((cum < kr).astype(jnp.int32))
    excl = cum - hist
    fi = (128 * lax.broadcasted_iota(jnp.int32, (r, 128), 0)
          + lax.broadcasted_iota(jnp.int32, (r, 128), 1))
    excl_at = jnp.sum(jnp.where(fi == jstar, excl, 0.0))
    new_kr = (kr - excl_at).astype(jnp.int32)
    new_prefix = prefix * mult + jstar
    ri = lax.broadcasted_iota(jnp.int32, (8, 128), 0)
    out_ref[...] = jnp.where(ri == 0, new_prefix,
                             jnp.where(ri == 1, new_kr, 0))


def _select(hist, mult, k_static, state):
    r = NB // 128
    h3 = hist.reshape(NW, r, 128)
    args = (h3,) if state is None else (h3, state)
    return pl.pallas_call(
        functools.partial(_select_body, mult, k_static, state is not None),
        out_shape=jax.ShapeDtypeStruct((8, 128), jnp.int32),
    )(*args)


def _mask_body(key_ref, w_ref, s_ref, o_ref):
    key = key_ref[0, 0]
    bits = jnp.where(key < 0, key & jnp.int32(0x7FFFFFFF), ~key)
    bits_v = jnp.zeros((1, 1), jnp.int32) + bits
    thr = lax.bitcast_convert_type(bits_v, jnp.float32)
    o_ref[...] = jnp.where(s_ref[...] < thr, 0.0, w_ref[...])


def _mask(key, weight, scores):
    rows, cols = scores.shape
    blk = 128
    return pl.pallas_call(
        _mask_body,
        grid=(rows // blk,),
        in_specs=[
            pl.BlockSpec(memory_space=pltpu.SMEM),
            pl.BlockSpec((blk, cols), lambda i: (i, 0)),
            pl.BlockSpec((blk, cols), lambda i: (i, 0)),
        ],
        out_specs=pl.BlockSpec((blk, cols), lambda i: (i, 0)),
        out_shape=jax.ShapeDtypeStruct(scores.shape, jnp.float32),
    )(key, weight, scores)


def kernel(weight, scores):
    n = scores.size
    k = int(1 + round(0.9 * (n - 1)))
    h1 = _hist_pass(scores, None)
    st1 = _select(h1, 1, k, None)
    h2 = _hist_pass(scores, st1)
    st2 = _select(h2, NB, None, st1)
    key = lax.slice(st2, (0, 0), (1, 1))
    return _mask(key, weight, scores)
